# Initial kernel scaffold; baseline (speedup 1.0000x reference)
import jax
import jax.numpy as jnp
from jax.experimental import pallas as pl


def kernel(x, edge_index, W0, b0, g0, be0, W1, b1, g1, be1, W2, b2, g2, be2, Wc, bc):
    raise NotImplementedError("write your pallas kernel here")


# speedup vs baseline: 11.6363x; 11.6363x over previous
import functools

import jax
import jax.numpy as jnp
from jax import lax
from jax.experimental import pallas as pl
from jax.experimental.pallas import tpu as pltpu
from jax.experimental.pallas import tpu_sc as plsc

N = 10000
D = 128
N_CLASS = 40
NC = 2
NS = 16
NTILES = NC * NS
CHUNK = 128
N_PAD = 10240
RPT = N_PAD // NS
PAD_ROW = N

_mesh = plsc.VectorSubcoreMesh(core_axis_name="c", subcore_axis_name="s")


def _deg_body(dst_hbm, out_hbm, idx_v, pay_v, shared_deg):
    c = lax.axis_index("c")
    s = lax.axis_index("s")
    tid = c * NS + s
    cpt = dst_hbm.shape[1]

    def _zero_row(i, carry):
        pay_v[i, :] = jnp.zeros((16,), jnp.float32)
        return carry
    lax.fori_loop(0, CHUNK, _zero_row, 0)
    for r in range(RPT // CHUNK):
        pltpu.sync_copy(pay_v, shared_deg.at[pl.ds(s * RPT + r * CHUNK, CHUNK)])

    def _one_row(i, carry):
        pay_v[i, :] = jnp.ones((16,), jnp.float32)
        return carry
    lax.fori_loop(0, CHUNK, _one_row, 0)

    pltpu.sync_copy(dst_hbm.at[tid], idx_v)
    plsc.subcore_barrier()

    def _scat(j, carry):
        pltpu.sync_copy(pay_v, shared_deg.at[idx_v.at[j]], add=True)
        return carry
    lax.fori_loop(0, cpt, _scat, 0)

    plsc.subcore_barrier()
    for r in range(RPT // CHUNK):
        sl = pl.ds(s * RPT + r * CHUNK, CHUNK)
        pltpu.sync_copy(shared_deg.at[sl], out_hbm.at[c, sl])


def _deg_call(dst_p):
    cpt = dst_p.shape[1]
    k = functools.partial(
        pl.kernel,
        mesh=_mesh,
        out_type=jax.ShapeDtypeStruct((NC, N_PAD, 16), jnp.float32),
        scratch_types=[
            pltpu.VMEM((cpt, CHUNK), jnp.int32),
            pltpu.VMEM((CHUNK, 16), jnp.float32),
            pltpu.VMEM_SHARED((N_PAD, 16), jnp.float32),
        ],
    )(_deg_body)
    return k(dst_p)


def _agg_body(h_hbm, src_hbm, dst_hbm, out_hbm, srcv, dstv, buf0, acc, sem):
    c = lax.axis_index("c")
    s = lax.axis_index("s")
    tid = c * NS + s
    cpt = src_hbm.shape[1]

    def _zero_row(i, carry):
        for kk in range(D // 16):
            buf0[i, pl.ds(kk * 16, 16)] = jnp.zeros((16,), jnp.float32)
        return carry
    lax.fori_loop(0, CHUNK, _zero_row, 0)
    for r in range(RPT // CHUNK):
        pltpu.sync_copy(buf0, acc.at[pl.ds(s * RPT + r * CHUNK, CHUNK)])

    pltpu.sync_copy(src_hbm.at[tid], srcv)
    pltpu.sync_copy(dst_hbm.at[tid], dstv)
    plsc.subcore_barrier()

    def _edge_chunk(j, carry):
        pltpu.async_copy(h_hbm.at[srcv.at[j]], buf0, sem).wait()
        pltpu.sync_copy(buf0, acc.at[dstv.at[j]], add=True)
        return carry
    lax.fori_loop(0, cpt, _edge_chunk, 0)

    plsc.subcore_barrier()
    for r in range(RPT // CHUNK):
        sl = pl.ds(s * RPT + r * CHUNK, CHUNK)
        pltpu.sync_copy(acc.at[sl], out_hbm.at[c, sl])


def _agg_call(hp, src_p, dst_p):
    cpt = src_p.shape[1]
    k = functools.partial(
        pl.kernel,
        mesh=_mesh,
        out_type=jax.ShapeDtypeStruct((NC, N_PAD, D), jnp.float32),
        scratch_types=[
            pltpu.VMEM((cpt, CHUNK), jnp.int32),
            pltpu.VMEM((cpt, CHUNK), jnp.int32),
            pltpu.VMEM((CHUNK, D), jnp.float32),
            pltpu.VMEM_SHARED((N_PAD, D), jnp.float32),
            pltpu.SemaphoreType.DMA,
        ],
    )(_agg_body)
    return k(hp, src_p, dst_p)


def _tc1_body(degp_ref, x_ref, w_ref, hp_ref, dis_ref):
    deg = jnp.max(degp_ref[0] + degp_ref[1], axis=1, keepdims=True) + 1.0
    dis = lax.rsqrt(deg)
    dis_ref[...] = dis
    t = jnp.dot(x_ref[...], w_ref[...], preferred_element_type=jnp.float32)
    hp_ref[:N, :] = t * dis[:N, :]
    hp_ref[N:, :] = jnp.zeros((N_PAD - N, D), jnp.float32)


def _tc1_call(degp, x, w0):
    return pl.pallas_call(
        _tc1_body,
        out_shape=[
            jax.ShapeDtypeStruct((N_PAD, D), jnp.float32),
            jax.ShapeDtypeStruct((N_PAD, 1), jnp.float32),
        ],
    )(degp, x, w0)


def _bn_relu(agg_ref, hp_ref, dis_ref, b_ref, g_ref, be_ref):
    a = agg_ref[0, :N, :] + agg_ref[1, :N, :] + hp_ref[:N, :]
    sarr = a * dis_ref[:N, :] + b_ref[...]
    mean = jnp.mean(sarr, axis=0, keepdims=True)
    xc = sarr - mean
    var = jnp.mean(xc * xc, axis=0, keepdims=True)
    y = xc * lax.rsqrt(var + 1e-5) * g_ref[...] + be_ref[...]
    return jnp.maximum(y, 0.0)


def _ep_mid_body(agg_ref, hp_ref, dis_ref, b_ref, g_ref, be_ref, wn_ref,
                 out_ref):
    r = _bn_relu(agg_ref, hp_ref, dis_ref, b_ref, g_ref, be_ref)
    t = jnp.dot(r, wn_ref[...], preferred_element_type=jnp.float32)
    out_ref[:N, :] = t * dis_ref[:N, :]
    out_ref[N:, :] = jnp.zeros((N_PAD - N, D), jnp.float32)


def _ep_mid_call(agg, hp, dis, b, g, be, wn):
    return pl.pallas_call(
        _ep_mid_body,
        out_shape=jax.ShapeDtypeStruct((N_PAD, D), jnp.float32),
    )(agg, hp, dis, b, g, be, wn)


def _ep_last_body(agg_ref, hp_ref, dis_ref, b_ref, g_ref, be_ref, wc_ref,
                  bc_ref, out_ref):
    r = _bn_relu(agg_ref, hp_ref, dis_ref, b_ref, g_ref, be_ref)
    out_ref[...] = jnp.dot(r, wc_ref[...],
                           preferred_element_type=jnp.float32) + bc_ref[...]


def _ep_last_call(agg, hp, dis, b, g, be, wc, bc):
    return pl.pallas_call(
        _ep_last_body,
        out_shape=jax.ShapeDtypeStruct((N, N_CLASS), jnp.float32),
    )(agg, hp, dis, b, g, be, wc, bc)


def kernel(x, edge_index, W0, b0, g0, be0, W1, b1, g1, be1, W2, b2, g2, be2,
           Wc, bc):
    E = edge_index.shape[1]
    cpt = -(-E // (NTILES * CHUNK))
    epad = NTILES * CHUNK * cpt
    pad = jnp.full((epad - E,), PAD_ROW, jnp.int32)
    src_p = jnp.concatenate([edge_index[0], pad]).reshape(NTILES, cpt, CHUNK)
    dst_p = jnp.concatenate([edge_index[1], pad]).reshape(NTILES, cpt, CHUNK)

    degp = _deg_call(dst_p)
    hp, dis = _tc1_call(degp, x, W0)

    agg = _agg_call(hp, src_p, dst_p)
    hp = _ep_mid_call(agg, hp, dis, b0.reshape(1, D), g0.reshape(1, D),
                      be0.reshape(1, D), W1)
    agg = _agg_call(hp, src_p, dst_p)
    hp = _ep_mid_call(agg, hp, dis, b1.reshape(1, D), g1.reshape(1, D),
                      be1.reshape(1, D), W2)
    agg = _agg_call(hp, src_p, dst_p)
    return _ep_last_call(agg, hp, dis, b2.reshape(1, D), g2.reshape(1, D),
                         be2.reshape(1, D), Wc, bc.reshape(1, N_CLASS))
